# baseline (speedup 1.0000x reference)
import functools

import jax
import jax.numpy as jnp
from jax import lax
from jax.experimental import pallas as pl
from jax.experimental.pallas import tpu as pltpu
from jax.experimental.pallas import tpu_sc as plsc

_NC = 2
_NS = 16
_NW = _NC * _NS
_CHUNK = 128
_LANES = 16


def _emb_body(nchunk, bsz, seq_len, dim, ids_hbm, table_hbm, pos_hbm, out_hbm,
              idx_v, buf0, buf1, pos_v, gsem0, gsem1, osem0, osem1):
    c = lax.axis_index("c")
    s = lax.axis_index("s")
    wid = s * _NC + c
    pos_per_w = seq_len // _NW
    batches_per_chunk = _CHUNK // pos_per_w

    pltpu.sync_copy(ids_hbm.at[wid], idx_v)
    p0 = wid * pos_per_w
    pltpu.sync_copy(pos_hbm.at[pl.ds(p0, pos_per_w)], pos_v)

    bufs = (buf0, buf1)
    gsems = (gsem0, gsem1)
    osems = (osem0, osem1)

    gathers = [None, None]
    out_copies = [[], []]

    gathers[0] = pltpu.async_copy(table_hbm.at[idx_v.at[0]], bufs[0], gsems[0])

    for cidx in range(nchunk):
        p = cidx % 2
        buf = bufs[p]
        gathers[p].wait()
        if cidx + 1 < nchunk:
            for cp in out_copies[1 - p]:
                cp.wait()
            out_copies[1 - p] = []
            gathers[1 - p] = pltpu.async_copy(
                table_hbm.at[idx_v.at[cidx + 1]], bufs[1 - p], gsems[1 - p])

        def add_row(r, carry, buf=buf):
            pr = lax.rem(r, pos_per_w)
            for d in range(dim // _LANES):
                v = pos_v[pr, pl.ds(d * _LANES, _LANES)]
                plsc.addupdate(buf.at[r, pl.ds(d * _LANES, _LANES)], v)
            return carry

        lax.fori_loop(0, _CHUNK, add_row, 0)

        b_base = cidx * batches_per_chunk
        for k in range(batches_per_chunk):
            dst = out_hbm.at[pl.ds((b_base + k) * seq_len + p0, pos_per_w)]
            src = buf.at[pl.ds(k * pos_per_w, pos_per_w)]
            out_copies[p].append(pltpu.async_copy(src, dst, osems[p]))

    for cp in out_copies[0] + out_copies[1]:
        cp.wait()


def kernel(input_ids, char_table, pos_table):
    bsz, seq_len = input_ids.shape
    vocab, dim = char_table.shape
    total = bsz * seq_len
    pos_per_w = seq_len // _NW
    rows_per_w = total // _NW
    nchunk = rows_per_w // _CHUNK

    ids3 = (input_ids.reshape(bsz, _NW, pos_per_w)
            .transpose(1, 0, 2)
            .reshape(_NW, nchunk, _CHUNK))

    mesh = plsc.VectorSubcoreMesh(core_axis_name="c", subcore_axis_name="s")
    body = functools.partial(_emb_body, nchunk, bsz, seq_len, dim)
    out = pl.kernel(
        body,
        out_type=jax.ShapeDtypeStruct((total, dim), jnp.float32),
        mesh=mesh,
        scratch_types=[
            pltpu.VMEM((nchunk, _CHUNK), jnp.int32),
            pltpu.VMEM((_CHUNK, dim), jnp.float32),
            pltpu.VMEM((_CHUNK, dim), jnp.float32),
            pltpu.VMEM((pos_per_w, dim), jnp.float32),
            pltpu.SemaphoreType.DMA,
            pltpu.SemaphoreType.DMA,
            pltpu.SemaphoreType.DMA,
            pltpu.SemaphoreType.DMA,
        ],
    )(ids3, char_table, pos_table)
    return out.reshape(bsz, seq_len, dim)


# speedup vs baseline: 1.0060x; 1.0060x over previous
import functools

import jax
import jax.numpy as jnp
from jax import lax
from jax.experimental import pallas as pl
from jax.experimental.pallas import tpu as pltpu
from jax.experimental.pallas import tpu_sc as plsc

_NC = 2
_NS = 16
_NW = _NC * _NS
_PB = 16
_BB = 2
_CHUNK = 128
_NBUF = 3
_LANES = 16


def _emb_body(nchunk, bsz, seq_len, dim, ids_hbm, table_hbm, pos_hbm, out_hbm,
              idx_v, buf0, buf1, buf2, pos_v, gs0, gs1, gs2, os0, os1, os2):
    c = lax.axis_index("c")
    s = lax.axis_index("s")
    wid = s * _NC + c
    pos_per_w = seq_len // _PB
    batches_per_chunk = _CHUNK // pos_per_w
    pb = wid // _BB
    bhalf = lax.rem(wid, _BB)
    p0 = pb * pos_per_w
    b0 = bhalf * (bsz // _BB)

    pltpu.sync_copy(ids_hbm.at[wid], idx_v)
    pltpu.sync_copy(pos_hbm.at[pl.ds(p0, pos_per_w)], pos_v)

    bufs = (buf0, buf1, buf2)
    gsems = (gs0, gs1, gs2)
    osems = (os0, os1, os2)

    gathers = [None] * _NBUF
    out_copies = [[] for _ in range(_NBUF)]

    for c0 in range(min(_NBUF - 1, nchunk)):
        gathers[c0] = pltpu.async_copy(
            table_hbm.at[idx_v.at[c0]], bufs[c0], gsems[c0])

    for cidx in range(nchunk):
        p = cidx % _NBUF
        buf = bufs[p]
        gathers[p].wait()
        nxt = cidx + _NBUF - 1
        if nxt < nchunk:
            q = nxt % _NBUF
            for cp in out_copies[q]:
                cp.wait()
            out_copies[q] = []
            gathers[q] = pltpu.async_copy(
                table_hbm.at[idx_v.at[nxt]], bufs[q], gsems[q])

        def add_row(r, carry, buf=buf):
            pr = lax.rem(r, pos_per_w)
            for d in range(dim // _LANES):
                v = pos_v[pr, pl.ds(d * _LANES, _LANES)]
                plsc.addupdate(buf.at[r, pl.ds(d * _LANES, _LANES)], v)
            return carry

        lax.fori_loop(0, _CHUNK, add_row, 0)

        for k in range(batches_per_chunk):
            b = b0 + cidx * batches_per_chunk + k
            dst = out_hbm.at[pl.ds(b * seq_len + p0, pos_per_w)]
            src = buf.at[pl.ds(k * pos_per_w, pos_per_w)]
            out_copies[p].append(pltpu.async_copy(src, dst, osems[p]))

    for lst in out_copies:
        for cp in lst:
            cp.wait()


def kernel(input_ids, char_table, pos_table):
    bsz, seq_len = input_ids.shape
    vocab, dim = char_table.shape
    total = bsz * seq_len
    pos_per_w = seq_len // _PB
    rows_per_w = total // _NW
    nchunk = rows_per_w // _CHUNK

    ids3 = (input_ids.reshape(_BB, bsz // _BB, _PB, pos_per_w)
            .transpose(2, 0, 1, 3)
            .reshape(_NW, nchunk, _CHUNK))

    mesh = plsc.VectorSubcoreMesh(core_axis_name="c", subcore_axis_name="s")
    body = functools.partial(_emb_body, nchunk, bsz, seq_len, dim)
    out = pl.kernel(
        body,
        out_type=jax.ShapeDtypeStruct((total, dim), jnp.float32),
        mesh=mesh,
        scratch_types=[
            pltpu.VMEM((nchunk, _CHUNK), jnp.int32),
            pltpu.VMEM((_CHUNK, dim), jnp.float32),
            pltpu.VMEM((_CHUNK, dim), jnp.float32),
            pltpu.VMEM((_CHUNK, dim), jnp.float32),
            pltpu.VMEM((pos_per_w, dim), jnp.float32),
            pltpu.SemaphoreType.DMA,
            pltpu.SemaphoreType.DMA,
            pltpu.SemaphoreType.DMA,
            pltpu.SemaphoreType.DMA,
            pltpu.SemaphoreType.DMA,
            pltpu.SemaphoreType.DMA,
        ],
    )(ids3, char_table, pos_table)
    return out.reshape(bsz, seq_len, dim)


# speedup vs baseline: 1.4383x; 1.4298x over previous
import functools

import jax
import jax.numpy as jnp
from jax import lax
from jax.experimental import pallas as pl
from jax.experimental.pallas import tpu as pltpu
from jax.experimental.pallas import tpu_sc as plsc

_NC = 2
_NS = 16
_NW = _NC * _NS
_CHUNK = 128
_NBUF = 3
_LANES = 16


def _emb_body(nchunk, bsz, seq_len, dim, ids_hbm, table_hbm, pos_hbm, out_hbm,
              idx_v, buf0, buf1, buf2, pos_v, pos_sh,
              gs0, gs1, gs2, os0, os1, os2):
    c = lax.axis_index("c")
    s = lax.axis_index("s")
    wid = s * _NC + c
    base = wid * nchunk * _CHUNK

    pltpu.sync_copy(ids_hbm.at[wid], idx_v)

    @pl.when(s == 0)
    def _():
        pltpu.sync_copy(pos_hbm, pos_sh)

    plsc.subcore_barrier()
    pltpu.sync_copy(pos_sh, pos_v)

    bufs = (buf0, buf1, buf2)
    gsems = (gs0, gs1, gs2)
    osems = (os0, os1, os2)

    gathers = [None] * _NBUF
    out_copies = [None] * _NBUF

    for c0 in range(min(_NBUF - 1, nchunk)):
        gathers[c0] = pltpu.async_copy(
            table_hbm.at[idx_v.at[c0]], bufs[c0], gsems[c0])

    for cidx in range(nchunk):
        p = cidx % _NBUF
        buf = bufs[p]
        gathers[p].wait()
        nxt = cidx + _NBUF - 1
        if nxt < nchunk:
            q = nxt % _NBUF
            if out_copies[q] is not None:
                out_copies[q].wait()
                out_copies[q] = None
            gathers[q] = pltpu.async_copy(
                table_hbm.at[idx_v.at[nxt]], bufs[q], gsems[q])

        pos_base = (cidx * _CHUNK) % seq_len

        def add_row(r, carry, buf=buf, pos_base=pos_base):
            for d in range(dim // _LANES):
                v = pos_v[pos_base + r, pl.ds(d * _LANES, _LANES)]
                plsc.addupdate(buf.at[r, pl.ds(d * _LANES, _LANES)], v)
            return carry

        lax.fori_loop(0, _CHUNK, add_row, 0)

        out_copies[p] = pltpu.async_copy(
            buf, out_hbm.at[pl.ds(base + cidx * _CHUNK, _CHUNK)], osems[p])

    for cp in out_copies:
        if cp is not None:
            cp.wait()


def kernel(input_ids, char_table, pos_table):
    bsz, seq_len = input_ids.shape
    vocab, dim = char_table.shape
    total = bsz * seq_len
    rows_per_w = total // _NW
    nchunk = rows_per_w // _CHUNK

    ids3 = input_ids.reshape(_NW, nchunk, _CHUNK)

    mesh = plsc.VectorSubcoreMesh(core_axis_name="c", subcore_axis_name="s")
    body = functools.partial(_emb_body, nchunk, bsz, seq_len, dim)
    out = pl.kernel(
        body,
        out_type=jax.ShapeDtypeStruct((total, dim), jnp.float32),
        mesh=mesh,
        scratch_types=[
            pltpu.VMEM((nchunk, _CHUNK), jnp.int32),
            pltpu.VMEM((_CHUNK, dim), jnp.float32),
            pltpu.VMEM((_CHUNK, dim), jnp.float32),
            pltpu.VMEM((_CHUNK, dim), jnp.float32),
            pltpu.VMEM((seq_len, dim), jnp.float32),
            pltpu.VMEM_SHARED((seq_len, dim), jnp.float32),
            pltpu.SemaphoreType.DMA,
            pltpu.SemaphoreType.DMA,
            pltpu.SemaphoreType.DMA,
            pltpu.SemaphoreType.DMA,
            pltpu.SemaphoreType.DMA,
            pltpu.SemaphoreType.DMA,
        ],
    )(ids3, char_table, pos_table)
    return out.reshape(bsz, seq_len, dim)
